# baseline (speedup 1.0000x reference)
import functools

import jax
import jax.numpy as jnp
from jax import lax
from jax.experimental import pallas as pl
from jax.experimental.pallas import tpu as pltpu
from jax.experimental.pallas import tpu_sc as plsc

V = 1000000
F = 26
D = 16
B = 4096
H = 400

_INFO = plsc.get_sparse_core_info()
NC = _INFO.num_cores
NS = _INFO.num_subcores
NW = NC * NS
N = B * F
PER_W = N // NW
CHUNK = 128
NCH = PER_W // CHUNK

TC_BLK = 16384


def _transpose_body(wt_ref, o_ref):
    eye = jnp.eye(D, dtype=jnp.float32)
    o_ref[...] = lax.dot_general(
        wt_ref[...], eye, (((0,), (0,)), ((), ())),
        preferred_element_type=jnp.float32)


def _transpose_table(wt):
    grid = (pl.cdiv(V, TC_BLK),)
    return pl.pallas_call(
        _transpose_body,
        grid=grid,
        in_specs=[pl.BlockSpec((D, TC_BLK), lambda i: (0, i))],
        out_specs=pl.BlockSpec((TC_BLK, D), lambda i: (i, 0)),
        out_shape=jax.ShapeDtypeStruct((V, D), jnp.float32),
    )(wt)


def _make_sc_gather():
    mesh = plsc.VectorSubcoreMesh(core_axis_name="c", subcore_axis_name="s")

    @functools.partial(
        pl.kernel,
        mesh=mesh,
        compiler_params=pltpu.CompilerParams(use_tc_tiling_on_sc=False),
        out_type=[
            jax.ShapeDtypeStruct((N, D), jnp.float32),
            jax.ShapeDtypeStruct((N,), jnp.float32),
        ],
        scratch_types=[
            pltpu.VMEM((NCH, CHUNK), jnp.int32),
            pltpu.VMEM((PER_W, D), jnp.float32),
            pltpu.VMEM((PER_W,), jnp.float32),
            pltpu.SemaphoreType.DMA,
            pltpu.SemaphoreType.DMA,
        ],
    )
    def gather_k(idx_hbm, vtab_hbm, wtab_hbm, rows_out, w_out,
                 idx_v, rows_v, w_v, sem_v, sem_w):
        wid = lax.axis_index("s") * NC + lax.axis_index("c")
        base = wid * PER_W
        pltpu.sync_copy(idx_hbm.at[wid], idx_v)

        def body(j, carry):
            cp1 = pltpu.async_copy(
                vtab_hbm.at[idx_v.at[j]], rows_v.at[pl.ds(j * CHUNK, CHUNK)],
                sem_v)
            cp2 = pltpu.async_copy(
                wtab_hbm.at[idx_v.at[j]], w_v.at[pl.ds(j * CHUNK, CHUNK)],
                sem_w)
            cp1.wait()
            cp2.wait()
            return carry

        lax.fori_loop(0, NCH, body, 0)
        pltpu.sync_copy(rows_v, rows_out.at[pl.ds(base, PER_W)])
        pltpu.sync_copy(w_v, w_out.at[pl.ds(base, PER_W)])

    return gather_k


_sc_gather = _make_sc_gather()


def _mlp_body(x_ref, w_ref, s_ref, d0_ref, b0_ref, d1_ref, b1_ref,
              d2_ref, b2_ref, o_ref):
    x = x_ref[...]
    o1 = jnp.sum(w_ref[...], axis=1, keepdims=True)
    sum_sq = jnp.sum(x * x, axis=1, keepdims=True)
    sv = jnp.dot(x, s_ref[...], preferred_element_type=jnp.float32)
    o2 = jnp.sum(sv * sv, axis=1, keepdims=True) - sum_sq
    h = jnp.dot(x, d0_ref[...], preferred_element_type=jnp.float32) + b0_ref[...]
    h = jnp.maximum(h, 0.0)
    h = jnp.dot(h, d1_ref[...], preferred_element_type=jnp.float32) + b1_ref[...]
    h = jnp.maximum(h, 0.0)
    o3 = jnp.dot(h, d2_ref[...], preferred_element_type=jnp.float32) + b2_ref[...]
    o_ref[...] = jax.nn.sigmoid(o1 + o2 + o3)


def kernel(inputs, W_weight, W_vector, D0, b0, D1, b1, D2, b2):
    idx = inputs.astype(jnp.int32)
    idx3 = idx.reshape(NW, NCH, CHUNK)

    vtab = _transpose_table(W_vector.T)
    wflat = W_weight.reshape(V)
    rows, wg = _sc_gather(idx3, vtab, wflat)
    x = rows.reshape(B, F * D)
    wg = wg.reshape(B, F)

    s_mat = jax.nn.one_hot(jnp.arange(F * D) % D, D, dtype=jnp.float32)

    BB = 1024
    grid = (B // BB,)
    out = pl.pallas_call(
        _mlp_body,
        grid=grid,
        in_specs=[
            pl.BlockSpec((BB, F * D), lambda i: (i, 0)),
            pl.BlockSpec((BB, F), lambda i: (i, 0)),
            pl.BlockSpec((F * D, D), lambda i: (0, 0)),
            pl.BlockSpec((F * D, H), lambda i: (0, 0)),
            pl.BlockSpec((1, H), lambda i: (0, 0)),
            pl.BlockSpec((H, H), lambda i: (0, 0)),
            pl.BlockSpec((1, H), lambda i: (0, 0)),
            pl.BlockSpec((H, 1), lambda i: (0, 0)),
            pl.BlockSpec((1, 1), lambda i: (0, 0)),
        ],
        out_specs=pl.BlockSpec((BB, 1), lambda i: (i, 0)),
        out_shape=jax.ShapeDtypeStruct((B, 1), jnp.float32),
    )(x, wg, s_mat, D0, b0.reshape(1, H), D1, b1.reshape(1, H),
      D2, b2.reshape(1, 1))
    return out


# speedup vs baseline: 1.1972x; 1.1972x over previous
import functools

import jax
import jax.numpy as jnp
from jax import lax
from jax.experimental import pallas as pl
from jax.experimental.pallas import tpu as pltpu
from jax.experimental.pallas import tpu_sc as plsc

V = 1000000
F = 26
D = 16
B = 4096
H = 400

_INFO = plsc.get_sparse_core_info()
NC = _INFO.num_cores
NS = _INFO.num_subcores
NW = NC * NS
N = B * F
PER_W = N // NW
CHUNK = 128
NCH = PER_W // CHUNK

TC_BLK = 16384


def _transpose_body(wt_ref, o_ref):
    eye = jnp.eye(D, dtype=jnp.float32)
    o_ref[...] = lax.dot_general(
        wt_ref[...], eye, (((0,), (0,)), ((), ())),
        preferred_element_type=jnp.float32)


def _transpose_table(wt):
    grid = (pl.cdiv(V, TC_BLK),)
    return pl.pallas_call(
        _transpose_body,
        grid=grid,
        in_specs=[pl.BlockSpec((D, TC_BLK), lambda i: (0, i))],
        out_specs=pl.BlockSpec((TC_BLK, D), lambda i: (i, 0)),
        out_shape=jax.ShapeDtypeStruct((V, D), jnp.float32),
    )(wt)


def _make_sc_gather():
    mesh = plsc.VectorSubcoreMesh(core_axis_name="c", subcore_axis_name="s")

    @functools.partial(
        pl.kernel,
        mesh=mesh,
        compiler_params=pltpu.CompilerParams(use_tc_tiling_on_sc=False),
        out_type=[
            jax.ShapeDtypeStruct((N, D), jnp.float32),
            jax.ShapeDtypeStruct((N,), jnp.float32),
        ],
        scratch_types=[
            pltpu.VMEM((NCH, CHUNK), jnp.int32),
            pltpu.VMEM((PER_W, D), jnp.float32),
            pltpu.VMEM((PER_W,), jnp.float32),
            pltpu.SemaphoreType.DMA,
            pltpu.SemaphoreType.DMA,
        ],
    )
    def gather_k(idx_hbm, vtab_hbm, wtab_hbm, rows_out, w_out,
                 idx_v, rows_v, w_v, sem_v, sem_w):
        wid = lax.axis_index("s") * NC + lax.axis_index("c")
        base = wid * PER_W
        pltpu.sync_copy(idx_hbm.at[wid], idx_v)

        def body(j, carry):
            cp1 = pltpu.async_copy(
                vtab_hbm.at[idx_v.at[j]], rows_v.at[pl.ds(j * CHUNK, CHUNK)],
                sem_v)
            cp2 = pltpu.async_copy(
                wtab_hbm.at[idx_v.at[j]], w_v.at[pl.ds(j * CHUNK, CHUNK)],
                sem_w)
            cp1.wait()
            cp2.wait()
            return carry

        lax.fori_loop(0, NCH, body, 0)
        pltpu.sync_copy(rows_v, rows_out.at[pl.ds(base, PER_W)])
        pltpu.sync_copy(w_v, w_out.at[pl.ds(base, PER_W)])

    return gather_k


_sc_gather = _make_sc_gather()


def _mlp_body(x_ref, w_ref, s_ref, d0_ref, b0_ref, d1_ref, b1_ref,
              d2_ref, b2_ref, o_ref):
    x = x_ref[...]
    o1 = jnp.sum(w_ref[...], axis=1, keepdims=True)
    sum_sq = jnp.sum(x * x, axis=1, keepdims=True)
    sv = jnp.dot(x, s_ref[...], preferred_element_type=jnp.float32)
    o2 = jnp.sum(sv * sv, axis=1, keepdims=True) - sum_sq
    h = jnp.dot(x, d0_ref[...], preferred_element_type=jnp.float32) + b0_ref[...]
    h = jnp.maximum(h, 0.0)
    h = jnp.dot(h, d1_ref[...], preferred_element_type=jnp.float32) + b1_ref[...]
    h = jnp.maximum(h, 0.0)
    o3 = jnp.dot(h, d2_ref[...], preferred_element_type=jnp.float32) + b2_ref[...]
    o_ref[...] = jax.nn.sigmoid(o1 + o2 + o3)


def kernel(inputs, W_weight, W_vector, D0, b0, D1, b1, D2, b2):
    idx = inputs.astype(jnp.int32)
    idx3 = idx.reshape(NW, NCH, CHUNK)

    vtab = W_vector
    wflat = W_weight.reshape(V)
    rows, wg = _sc_gather(idx3, vtab, wflat)
    x = rows.reshape(B, F * D)
    wg = wg.reshape(B, F)

    s_mat = jax.nn.one_hot(jnp.arange(F * D) % D, D, dtype=jnp.float32)

    BB = 1024
    grid = (B // BB,)
    out = pl.pallas_call(
        _mlp_body,
        grid=grid,
        in_specs=[
            pl.BlockSpec((BB, F * D), lambda i: (i, 0)),
            pl.BlockSpec((BB, F), lambda i: (i, 0)),
            pl.BlockSpec((F * D, D), lambda i: (0, 0)),
            pl.BlockSpec((F * D, H), lambda i: (0, 0)),
            pl.BlockSpec((1, H), lambda i: (0, 0)),
            pl.BlockSpec((H, H), lambda i: (0, 0)),
            pl.BlockSpec((1, H), lambda i: (0, 0)),
            pl.BlockSpec((H, 1), lambda i: (0, 0)),
            pl.BlockSpec((1, 1), lambda i: (0, 0)),
        ],
        out_specs=pl.BlockSpec((BB, 1), lambda i: (i, 0)),
        out_shape=jax.ShapeDtypeStruct((B, 1), jnp.float32),
    )(x, wg, s_mat, D0, b0.reshape(1, H), D1, b1.reshape(1, H),
      D2, b2.reshape(1, 1))
    return out


# speedup vs baseline: 1.1998x; 1.0022x over previous
import functools

import jax
import jax.numpy as jnp
from jax import lax
from jax.experimental import pallas as pl
from jax.experimental.pallas import tpu as pltpu
from jax.experimental.pallas import tpu_sc as plsc

V = 1000000
F = 26
D = 16
B = 4096
H = 400

_INFO = plsc.get_sparse_core_info()
NC = _INFO.num_cores
NS = _INFO.num_subcores
NW = NC * NS
N = B * F
PER_W = N // NW
CHUNK = 128
NCH = PER_W // CHUNK

def _make_sc_gather():
    mesh = plsc.VectorSubcoreMesh(core_axis_name="c", subcore_axis_name="s")

    @functools.partial(
        pl.kernel,
        mesh=mesh,
        compiler_params=pltpu.CompilerParams(use_tc_tiling_on_sc=False),
        out_type=[
            jax.ShapeDtypeStruct((N, D), jnp.float32),
            jax.ShapeDtypeStruct((N,), jnp.float32),
        ],
        scratch_types=[
            pltpu.VMEM((NCH, CHUNK), jnp.int32),
            pltpu.VMEM((PER_W, D), jnp.float32),
            pltpu.VMEM((PER_W,), jnp.float32),
            pltpu.SemaphoreType.DMA,
            pltpu.SemaphoreType.DMA,
        ],
    )
    def gather_k(idx_hbm, vtab_hbm, wtab_hbm, rows_out, w_out,
                 idx_v, rows_v, w_v, sem_v, sem_w):
        wid = lax.axis_index("s") * NC + lax.axis_index("c")
        base = wid * PER_W
        pltpu.sync_copy(idx_hbm.at[wid], idx_v)

        def body(j, carry):
            cp1 = pltpu.async_copy(
                vtab_hbm.at[idx_v.at[j]], rows_v.at[pl.ds(j * CHUNK, CHUNK)],
                sem_v)
            cp2 = pltpu.async_copy(
                wtab_hbm.at[idx_v.at[j]], w_v.at[pl.ds(j * CHUNK, CHUNK)],
                sem_w)
            cp1.wait()
            cp2.wait()
            return carry

        lax.fori_loop(0, NCH, body, 0)
        pltpu.sync_copy(rows_v, rows_out.at[pl.ds(base, PER_W)])
        pltpu.sync_copy(w_v, w_out.at[pl.ds(base, PER_W)])

    return gather_k


_sc_gather = _make_sc_gather()


def _mlp_body(x_ref, w_ref, s_ref, d0_ref, b0_ref, d1_ref, b1_ref,
              d2_ref, b2_ref, o_ref):
    x = x_ref[...]
    o1 = jnp.sum(w_ref[...], axis=1, keepdims=True)
    sum_sq = jnp.sum(x * x, axis=1, keepdims=True)
    sv = jnp.dot(x, s_ref[...], preferred_element_type=jnp.float32)
    o2 = jnp.sum(sv * sv, axis=1, keepdims=True) - sum_sq
    h = jnp.dot(x, d0_ref[...], preferred_element_type=jnp.float32) + b0_ref[...]
    h = jnp.maximum(h, 0.0)
    h = jnp.dot(h, d1_ref[...], preferred_element_type=jnp.float32) + b1_ref[...]
    h = jnp.maximum(h, 0.0)
    o3 = jnp.dot(h, d2_ref[...], preferred_element_type=jnp.float32) + b2_ref[...]
    o_ref[...] = jax.nn.sigmoid(o1 + o2 + o3)


def kernel(inputs, W_weight, W_vector, D0, b0, D1, b1, D2, b2):
    idx = inputs.astype(jnp.int32)
    idx3 = idx.reshape(NW, NCH, CHUNK)

    vtab = lax.optimization_barrier(W_vector.reshape(V // 8, 8 * D)).reshape(V, D)
    wflat = W_weight.reshape(V)
    rows, wg = _sc_gather(idx3, vtab, wflat)
    x = rows.reshape(B, F * D)
    wg = wg.reshape(B, F)

    s_mat = jax.nn.one_hot(jnp.arange(F * D) % D, D, dtype=jnp.float32)

    BB = 1024
    grid = (B // BB,)
    out = pl.pallas_call(
        _mlp_body,
        grid=grid,
        in_specs=[
            pl.BlockSpec((BB, F * D), lambda i: (i, 0)),
            pl.BlockSpec((BB, F), lambda i: (i, 0)),
            pl.BlockSpec((F * D, D), lambda i: (0, 0)),
            pl.BlockSpec((F * D, H), lambda i: (0, 0)),
            pl.BlockSpec((1, H), lambda i: (0, 0)),
            pl.BlockSpec((H, H), lambda i: (0, 0)),
            pl.BlockSpec((1, H), lambda i: (0, 0)),
            pl.BlockSpec((H, 1), lambda i: (0, 0)),
            pl.BlockSpec((1, 1), lambda i: (0, 0)),
        ],
        out_specs=pl.BlockSpec((BB, 1), lambda i: (i, 0)),
        out_shape=jax.ShapeDtypeStruct((B, 1), jnp.float32),
    )(x, wg, s_mat, D0, b0.reshape(1, H), D1, b1.reshape(1, H),
      D2, b2.reshape(1, 1))
    return out
